# baseline (speedup 1.0000x reference)
import jax
import jax.numpy as jnp
from jax.experimental import pallas as pl
from jax.experimental.pallas import tpu as pltpu

B, U, C, DK, DV = 16, 64, 1024, 64, 64
RATE = 0.01
CP = C * DK // 128
UB = 4
NCH = U // UB
NBUF = 3
LOOKAHEAD = 2


def _in_copies(ch, s, w_hbm, mk_hbm, mv_hbm, wb, mkb, mvb, insem):
    cps = []
    for j in range(UB):
        cps.append(pltpu.make_async_copy(
            w_hbm.at[:, ch * UB + j, :], wb.at[s, j], insem.at[s, j]))
    cps.append(pltpu.make_async_copy(
        mk_hbm.at[pl.ds(ch * UB, UB)], mkb.at[s], insem.at[s, UB]))
    cps.append(pltpu.make_async_copy(
        mv_hbm.at[pl.ds(ch * UB, UB)], mvb.at[s], insem.at[s, UB + 1]))
    return cps


def _out_copies(ch, s, okb, ovb, ok_hbm, ov_hbm, outsem):
    return [
        pltpu.make_async_copy(okb.at[s], ok_hbm.at[pl.ds(ch * UB, UB)], outsem.at[s, 0]),
        pltpu.make_async_copy(ovb.at[s], ov_hbm.at[pl.ds(ch * UB, UB)], outsem.at[s, 1]),
    ]


def _body(w_hbm, knT_hbm, vnT_hbm, r_hbm, mk_hbm, mv_hbm, mr_hbm,
          ok_hbm, ov_hbm, or_hbm,
          wb, mkb, mvb, okb, ovb, knb, vnb, rb, mrb, orb,
          insem, outsem, psem, orsem):
    i = pl.program_id(0)
    s = i % NBUF

    pre = [
        pltpu.make_async_copy(knT_hbm, knb, psem.at[0]),
        pltpu.make_async_copy(vnT_hbm, vnb, psem.at[1]),
        pltpu.make_async_copy(r_hbm, rb, psem.at[2]),
        pltpu.make_async_copy(mr_hbm, mrb, psem.at[3]),
    ]

    @pl.when(i == 0)
    def _prologue():
        for cp in pre:
            cp.start()
        for ch in range(LOOKAHEAD + 1):
            for cp in _in_copies(ch, ch % NBUF, w_hbm, mk_hbm, mv_hbm,
                                 wb, mkb, mvb, insem):
                cp.start()
        for cp in pre:
            cp.wait()

    @pl.when((i > 0) & (i + LOOKAHEAD < NCH))
    def _issue_ahead():
        for cp in _in_copies(i + LOOKAHEAD, (i + LOOKAHEAD) % NBUF,
                             w_hbm, mk_hbm, mv_hbm, wb, mkb, mvb, insem):
            cp.start()

    for cp in _in_copies(i, s, w_hbm, mk_hbm, mv_hbm, wb, mkb, mvb, insem):
        cp.wait()

    @pl.when(i >= NBUF)
    def _drain_slot():
        for cp in _out_copies(i, s, okb, ovb, ok_hbm, ov_hbm, outsem):
            cp.wait()

    z = jnp.zeros((B, DK), jnp.float32)
    on = jnp.ones((B, DK), jnp.float32)
    r_col = rb[...]

    for j in range(UB):
        w_u = wb[s, j]
        fresh = w_u * RATE
        lhs = jnp.concatenate([fresh[:, :CP], fresh[:, CP:]], axis=0)
        kn_u = knb[i * UB + j]
        vn_u = vnb[i * UB + j]
        rhs = jnp.concatenate([
            jnp.concatenate([kn_u, z, vn_u, z, on, z], axis=1),
            jnp.concatenate([z, kn_u, z, vn_u, z, on], axis=1),
        ], axis=0)
        acc = jax.lax.dot_general(
            lhs, rhs, dimension_numbers=(((0,), (0,)), ((), ())),
            preferred_element_type=jnp.float32)
        retain2 = 1.0 - acc[:, 256:]
        okb[s, j] = mkb[s, j] * retain2 + acc[:, :128]
        ovb[s, j] = mvb[s, j] * retain2 + acc[:, 128:256]
        fs = jnp.sum(fresh, axis=0)
        rw = jnp.sum(fresh * r_col, axis=0)
        orb[i * UB + j, :] = mrb[i * UB + j, :] * (1.0 - fs) + rw

    for cp in _out_copies(i, s, okb, ovb, ok_hbm, ov_hbm, outsem):
        cp.start()

    @pl.when(i == NCH - 1)
    def _epilogue():
        or_cp = pltpu.make_async_copy(orb, or_hbm, orsem)
        or_cp.start()
        for back in range(NBUF):
            ch = NCH - 1 - back
            for cp in _out_copies(ch, ch % NBUF, okb, ovb, ok_hbm, ov_hbm, outsem):
                cp.wait()
        or_cp.wait()


def kernel(weights, key_new, value_new, reward, mem_keys, mem_values, mem_rewards):
    wD = weights.reshape(B, U, CP, 2).transpose(0, 1, 3, 2).reshape(B, U, C)
    mrD = mem_rewards.reshape(U, CP, 2).transpose(0, 2, 1).reshape(U, C)
    knT = key_new.transpose(1, 0, 2)
    vnT = value_new.transpose(1, 0, 2)
    r2 = reward.reshape(B, 1)
    mk2 = mem_keys.reshape(U, CP, 128)
    mv2 = mem_values.reshape(U, CP, 128)
    any_spec = pl.BlockSpec(memory_space=pl.ANY)
    out_k, out_v, out_r = pl.pallas_call(
        _body,
        grid=(NCH,),
        in_specs=[any_spec] * 7,
        out_specs=[any_spec] * 3,
        out_shape=[
            jax.ShapeDtypeStruct((U, CP, 128), jnp.float32),
            jax.ShapeDtypeStruct((U, CP, 128), jnp.float32),
            jax.ShapeDtypeStruct((U, C), jnp.float32),
        ],
        scratch_shapes=[
            pltpu.VMEM((NBUF, UB, B, C), jnp.float32),
            pltpu.VMEM((NBUF, UB, CP, 128), jnp.float32),
            pltpu.VMEM((NBUF, UB, CP, 128), jnp.float32),
            pltpu.VMEM((NBUF, UB, CP, 128), jnp.float32),
            pltpu.VMEM((NBUF, UB, CP, 128), jnp.float32),
            pltpu.VMEM((U, B, DK), jnp.float32),
            pltpu.VMEM((U, B, DV), jnp.float32),
            pltpu.VMEM((B, 1), jnp.float32),
            pltpu.VMEM((U, C), jnp.float32),
            pltpu.VMEM((U, C), jnp.float32),
            pltpu.SemaphoreType.DMA((NBUF, UB + 2)),
            pltpu.SemaphoreType.DMA((NBUF, 2)),
            pltpu.SemaphoreType.DMA((4,)),
            pltpu.SemaphoreType.DMA,
        ],
    )(wD, knT, vnT, r2, mk2, mv2, mrD)
    new_r = out_r.reshape(U, 2, CP).transpose(0, 2, 1).reshape(U, C)
    return out_k.reshape(U, C, DK), out_v.reshape(U, C, DV), new_r
